# baseline (speedup 1.0000x reference)
import functools

import jax
import jax.numpy as jnp
from jax import lax
from jax.experimental import pallas as pl
from jax.experimental.pallas import tpu as pltpu
from jax.experimental.pallas import tpu_sc as plsc

N = 10000
E = 320000
D = 128
BN_EPS = 1e-5

NC = 2
NS = 16
CH = 128
EPCP = 20480
EPAD = NS * EPCP - E
NCHUNK = EPCP // CH
NP = 10240
RPS = NP // NS
ZR = 80

@functools.cache
def _get_sc_segsum():
    mesh = plsc.VectorSubcoreMesh(
        core_axis_name="c", subcore_axis_name="s",
        num_cores=NC, num_subcores=NS)
    return functools.partial(
        pl.kernel,
        out_type=jax.ShapeDtypeStruct((NC, NP, D), jnp.float32),
        mesh=mesh,
        scratch_types=[
            pltpu.VMEM((2, CH), jnp.int32),
            pltpu.VMEM((2, CH), jnp.int32),
            pltpu.VMEM((CH, D), jnp.float32),
            pltpu.VMEM((CH, D), jnp.float32),
            pltpu.VMEM((ZR, D), jnp.float32),
            pltpu.VMEM_SHARED((NP, D), jnp.float32),
            pltpu.SemaphoreType.DMA,
            pltpu.SemaphoreType.DMA,
            pltpu.SemaphoreType.DMA,
            pltpu.SemaphoreType.DMA,
        ],
    )(_sc_segsum_body)


def _sc_segsum_body(src_hbm, ids_hbm, out_hbm,
                    ids0, ids1, gbuf0, gbuf1, zbuf, acc,
                    sg0, sg1, ss0, ss1):
    c = lax.axis_index("c")
    s = lax.axis_index("s")
    ids = (ids0, ids1)
    gbuf = (gbuf0, gbuf1)
    sg = (sg0, sg1)
    ss = (ss0, ss1)

    for r in range(ZR):
        for j in range(D // 16):
            zbuf[r, 16 * j:16 * (j + 1)] = jnp.zeros((16,), jnp.float32)
    for k in range(RPS // ZR):
        pltpu.sync_copy(zbuf, acc.at[pl.ds(s * RPS + k * ZR, ZR)])
    plsc.subcore_barrier()

    def load_idx(i, p):
        pltpu.sync_copy(ids_hbm.at[c, s, i], ids[p])

    def gather_desc(p):
        return pltpu.make_async_copy(src_hbm.at[ids[p].at[0]], gbuf[p], sg[p])

    def scatter_desc(p):
        return pltpu.make_async_copy(gbuf[p], acc.at[ids[p].at[1]], ss[p])

    load_idx(0, 0)
    gather_desc(0).start()
    gather_desc(0).wait()
    scatter_desc(0).start(add=True)
    load_idx(1, 1)
    gather_desc(1).start()

    def step(i, p):
        q = 1 - p
        gather_desc(p).wait()
        scatter_desc(p).start(add=True)
        scatter_desc(q).wait()
        load_idx(i + 1, q)
        gather_desc(q).start()

    def body(j, carry):
        step(2 * j + 1, 1)
        step(2 * j + 2, 0)
        return carry

    lax.fori_loop(0, NCHUNK // 2 - 1, body, 0)

    gather_desc(1).wait()
    scatter_desc(1).start(add=True)
    scatter_desc(0).wait()
    scatter_desc(1).wait()
    plsc.subcore_barrier()

    pltpu.sync_copy(acc.at[pl.ds(s * RPS, RPS)],
                    out_hbm.at[c, pl.ds(s * RPS, RPS)])


def _tc_body(x_ref, perb_ref, s0_ref, s1_ref,
             wo_ref, bo_ref, wt_ref, bt_ref,
             w1t_ref, b1_ref, gamma_ref, beta_ref, a_ref, w2t_ref, b2_ref,
             embed_ref, loss_ref):
    s0 = s0_ref[...]
    s01 = s0 + s1_ref[...]

    wo = wo_ref[...]
    bo = bo_ref[...]
    g1 = jnp.dot(s0, wo, preferred_element_type=jnp.float32) + bo
    g2 = jnp.dot(s01, wo, preferred_element_type=jnp.float32) + bo
    embed_ref[...] = x_ref[...] + perb_ref[...] + g2

    wt = wt_ref[...]
    bt = bt_ref[...]
    t1 = jnp.dot(s0, wt, preferred_element_type=jnp.float32) + bt
    t2 = jnp.dot(s01, wt, preferred_element_type=jnp.float32) + bt

    w1t = w1t_ref[...]
    w2t = w2t_ref[...]
    b1 = b1_ref[...]
    b2 = b2_ref[...]
    gamma = gamma_ref[...]
    beta = beta_ref[...]
    a = a_ref[0, 0]

    def predictor(z):
        h = jnp.dot(z, w1t, preferred_element_type=jnp.float32) + b1
        mean = jnp.mean(h, axis=0, keepdims=True)
        d = h - mean
        var = jnp.mean(d * d, axis=0, keepdims=True)
        h = gamma * d * jax.lax.rsqrt(var + BN_EPS) + beta
        h = jnp.where(h >= 0.0, h, a * h)
        return jnp.dot(h, w2t, preferred_element_type=jnp.float32) + b2

    p1 = predictor(g1)
    p2 = predictor(g2)

    def l2n(v):
        ss = jnp.sum(v * v, axis=-1, keepdims=True)
        return v / jnp.maximum(jnp.sqrt(ss), 1e-12)

    dots = jnp.sum(l2n(p1) * l2n(t2) + l2n(p2) * l2n(t1))
    loss_ref[0, 0] = 4.0 - 2.0 * dots / N


def _tc_stage(x, perb, s0, s1, W_online, b_online, W_target, b_target,
              W1, b1, gamma, beta, prelu_a, W2, b2):
    vmem = pl.BlockSpec(memory_space=pltpu.VMEM)
    smem = pl.BlockSpec(memory_space=pltpu.SMEM)
    embed, loss = pl.pallas_call(
        _tc_body,
        out_shape=[
            jax.ShapeDtypeStruct((N, D), jnp.float32),
            jax.ShapeDtypeStruct((1, 1), jnp.float32),
        ],
        in_specs=[vmem] * 12 + [smem] + [vmem] * 2,
        out_specs=[vmem, smem],
    )(x, perb, s0, s1,
      W_online, b_online.reshape(1, D), W_target, b_target.reshape(1, D),
      W1.T, b1.reshape(1, D), gamma.reshape(1, D), beta.reshape(1, D),
      prelu_a.reshape(1, 1), W2.T, b2.reshape(1, D))
    return embed, loss[0, 0]


def kernel(x, perb, edge_index, W_online, b_online, W_target, b_target,
           W1, b1, gamma, beta, prelu_a, W2, b2):
    row = edge_index[0]
    col = edge_index[1]
    src = jnp.concatenate([x, perb], axis=0)
    colp = jnp.concatenate([col, jnp.zeros((EPAD,), jnp.int32)])
    rowp = jnp.concatenate([row, jnp.full((EPAD,), N, jnp.int32)])
    cols = jnp.stack([colp, colp + N]).reshape(NC, NS, NCHUNK, CH)
    rows = jnp.broadcast_to(rowp.reshape(1, NS, NCHUNK, CH),
                            (NC, NS, NCHUNK, CH))
    ids = jnp.stack([cols, rows], axis=3)
    agg = _get_sc_segsum()(src, ids)[:, :N, :]
    embed, loss = _tc_stage(x, perb, agg[0], agg[1],
                            W_online, b_online, W_target, b_target,
                            W1, b1, gamma, beta, prelu_a, W2, b2)
    return (embed, loss)


# speedup vs baseline: 1.1868x; 1.1868x over previous
import functools

import jax
import jax.numpy as jnp
from jax import lax
from jax.experimental import pallas as pl
from jax.experimental.pallas import tpu as pltpu
from jax.experimental.pallas import tpu_sc as plsc

N = 10000
E = 320000
D = 128
DW = D // 2
BN_EPS = 1e-5

NC = 2
NS = 16
CH = 120
NCHUNK = 168
EPCP = NCHUNK * CH
EPAD = NS * EPCP - E
NP = 10240
RPS = NP // NS


@functools.cache
def _get_sc_segsum():
    mesh = plsc.VectorSubcoreMesh(
        core_axis_name="c", subcore_axis_name="s",
        num_cores=NC, num_subcores=NS)
    return functools.partial(
        pl.kernel,
        out_type=jax.ShapeDtypeStruct((NC, NP, D), jnp.float32),
        mesh=mesh,
        compiler_params=pltpu.CompilerParams(use_tc_tiling_on_sc=False),
        scratch_types=[
            pltpu.VMEM((4, 2, CH), jnp.int32),
            pltpu.VMEM((CH, DW), jnp.int32),
            pltpu.VMEM((CH, DW), jnp.int32),
            pltpu.VMEM((CH, D), jnp.float32),
            pltpu.VMEM((CH, D), jnp.float32),
            pltpu.VMEM_SHARED((NP, D), jnp.float32),
            pltpu.SemaphoreType.DMA,
            pltpu.SemaphoreType.DMA,
            pltpu.SemaphoreType.DMA,
            pltpu.SemaphoreType.DMA,
        ],
    )(_sc_segsum_body)


def _sc_segsum_body(src_hbm, ids_hbm, out_hbm,
                    ids, gbuf0, gbuf1, fbuf0, fbuf1, acc,
                    sg0, sg1, ss0, ss1):
    c = lax.axis_index("c")
    s = lax.axis_index("s")
    gbuf = (gbuf0, gbuf1)
    fbuf = (fbuf0, fbuf1)
    sg = (sg0, sg1)
    ss = (ss0, ss1)

    def zrow(r, carry):
        for j in range(D // 16):
            fbuf0[r, 16 * j:16 * (j + 1)] = jnp.zeros((16,), jnp.float32)
        return carry

    lax.fori_loop(0, CH, zrow, 0)
    for k in range(RPS // CH):
        pltpu.sync_copy(fbuf0, acc.at[pl.ds(s * RPS + k * CH, CH)])
    pltpu.sync_copy(fbuf0.at[pl.ds(0, RPS % CH)],
                    acc.at[pl.ds(s * RPS + (RPS // CH) * CH, RPS % CH)])
    plsc.subcore_barrier()

    def load_idx(i, r):
        pltpu.sync_copy(ids_hbm.at[c, s, i], ids.at[r])

    def gather_desc(p, r):
        return pltpu.make_async_copy(
            src_hbm.at[ids.at[r].at[0]], gbuf[p], sg[p])

    def scatter_desc(p, r):
        return pltpu.make_async_copy(
            fbuf[p], acc.at[ids.at[r].at[1]], ss[p])

    def convert(p):
        sh16 = jnp.broadcast_to(jnp.int32(16), (16,))
        mask = jnp.broadcast_to(jnp.int32(-65536), (16,))

        def crow(r, carry):
            for m in range(DW // 16):
                v = gbuf[p][r, 16 * m:16 * (m + 1)]
                fbuf[p][r, 32 * m:32 * m + 16] = lax.bitcast_convert_type(
                    lax.shift_left(v, sh16), jnp.float32)
                fbuf[p][r, 32 * m + 16:32 * m + 32] = lax.bitcast_convert_type(
                    lax.bitwise_and(v, mask), jnp.float32)
            return carry

        lax.fori_loop(0, CH, crow, 0)

    load_idx(0, 0)
    gather_desc(0, 0).start()
    load_idx(1, 1)
    gather_desc(1, 1).start()

    def step(i, islot, ss_wait, refill):
        p = islot % 2
        r = islot
        r2 = (islot + 2) % 4
        gather_desc(p, r).wait()
        if ss_wait:
            scatter_desc(p, r2).wait()
        convert(p)
        scatter_desc(p, r).start(add=True)
        if refill:
            load_idx(i + 2, r2)
            gather_desc(p, r2).start()

    step(0, 0, False, True)
    step(1, 1, False, True)

    def body(j, carry):
        for t in range(4):
            step(4 * j + 2 + t, (2 + t) % 4, True, True)
        return carry

    lax.fori_loop(0, (NCHUNK - 4) // 4, body, 0)

    step(NCHUNK - 2, (NCHUNK - 2) % 4, True, False)
    step(NCHUNK - 1, (NCHUNK - 1) % 4, True, False)
    scatter_desc(0, 2).wait()
    scatter_desc(1, 3).wait()
    plsc.subcore_barrier()

    pltpu.sync_copy(acc.at[pl.ds(s * RPS, RPS)],
                    out_hbm.at[c, pl.ds(s * RPS, RPS)])


def _tc_body(x_ref, perb_ref, s0_ref, s1_ref,
             wo_ref, bo_ref, wt_ref, bt_ref,
             w1t_ref, b1_ref, gamma_ref, beta_ref, a_ref, w2t_ref, b2_ref,
             embed_ref, loss_ref):
    s0 = s0_ref[...]
    s01 = s0 + s1_ref[...]

    wo = wo_ref[...]
    bo = bo_ref[...]
    g1 = jnp.dot(s0, wo, preferred_element_type=jnp.float32) + bo
    g2 = jnp.dot(s01, wo, preferred_element_type=jnp.float32) + bo
    embed_ref[...] = x_ref[...] + perb_ref[...] + g2

    wt = wt_ref[...]
    bt = bt_ref[...]
    t1 = jnp.dot(s0, wt, preferred_element_type=jnp.float32) + bt
    t2 = jnp.dot(s01, wt, preferred_element_type=jnp.float32) + bt

    w1t = w1t_ref[...]
    w2t = w2t_ref[...]
    b1 = b1_ref[...]
    b2 = b2_ref[...]
    gamma = gamma_ref[...]
    beta = beta_ref[...]
    a = a_ref[0, 0]

    def predictor(z):
        h = jnp.dot(z, w1t, preferred_element_type=jnp.float32) + b1
        mean = jnp.mean(h, axis=0, keepdims=True)
        d = h - mean
        var = jnp.mean(d * d, axis=0, keepdims=True)
        h = gamma * d * jax.lax.rsqrt(var + BN_EPS) + beta
        h = jnp.where(h >= 0.0, h, a * h)
        return jnp.dot(h, w2t, preferred_element_type=jnp.float32) + b2

    p1 = predictor(g1)
    p2 = predictor(g2)

    def l2n(v):
        ss = jnp.sum(v * v, axis=-1, keepdims=True)
        return v / jnp.maximum(jnp.sqrt(ss), 1e-12)

    dots = jnp.sum(l2n(p1) * l2n(t2) + l2n(p2) * l2n(t1))
    loss_ref[0, 0] = 4.0 - 2.0 * dots / N


def _tc_stage(x, perb, s0, s1, W_online, b_online, W_target, b_target,
              W1, b1, gamma, beta, prelu_a, W2, b2):
    vmem = pl.BlockSpec(memory_space=pltpu.VMEM)
    smem = pl.BlockSpec(memory_space=pltpu.SMEM)
    embed, loss = pl.pallas_call(
        _tc_body,
        out_shape=[
            jax.ShapeDtypeStruct((N, D), jnp.float32),
            jax.ShapeDtypeStruct((1, 1), jnp.float32),
        ],
        in_specs=[vmem] * 12 + [smem] + [vmem] * 2,
        out_specs=[vmem, smem],
    )(x, perb, s0, s1,
      W_online, b_online.reshape(1, D), W_target, b_target.reshape(1, D),
      W1.T, b1.reshape(1, D), gamma.reshape(1, D), beta.reshape(1, D),
      prelu_a.reshape(1, 1), W2.T, b2.reshape(1, D))
    return embed, loss[0, 0]


def kernel(x, perb, edge_index, W_online, b_online, W_target, b_target,
           W1, b1, gamma, beta, prelu_a, W2, b2):
    row = edge_index[0]
    col = edge_index[1]
    src = jnp.concatenate([x, perb], axis=0)
    u16 = jax.lax.bitcast_convert_type(src.astype(jnp.bfloat16), jnp.uint16)
    grp = u16.reshape(2 * N, D // 32, 2, 16)
    packed = jax.lax.bitcast_convert_type(
        (grp[:, :, 0, :].astype(jnp.uint32)
         | (grp[:, :, 1, :].astype(jnp.uint32) << 16)).reshape(2 * N, DW),
        jnp.int32)
    colp = jnp.concatenate([col, jnp.zeros((EPAD,), jnp.int32)])
    rowp = jnp.concatenate([row, jnp.full((EPAD,), N, jnp.int32)])
    cols = jnp.stack([colp, colp + N]).reshape(NC, NS, NCHUNK, CH)
    rows = jnp.broadcast_to(rowp.reshape(1, NS, NCHUNK, CH),
                            (NC, NS, NCHUNK, CH))
    ids = jnp.stack([cols, rows], axis=3)
    agg = _get_sc_segsum()(packed, ids)[:, :N, :]
    embed, loss = _tc_stage(x, perb, agg[0], agg[1],
                            W_online, b_online, W_target, b_target,
                            W1, b1, gamma, beta, prelu_a, W2, b2)
    return (embed, loss)
